# Initial kernel scaffold; baseline (speedup 1.0000x reference)
import jax
import jax.numpy as jnp
from jax.experimental import pallas as pl


def kernel(x, start_indices_L, start_indices_C, W1, b1, W2, b2):
    raise NotImplementedError("write your pallas kernel here")


# speedup vs baseline: 117.5366x; 117.5366x over previous
import functools

import jax
import jax.numpy as jnp
from jax.experimental import pallas as pl
from jax.experimental.pallas import tpu as pltpu

_PL, _PC = 8, 16
_FS = 100.0


def _mlp_tc_kernel(sl_ref, sc_ref, sl2_ref, x_ref, w1p_ref, w1t_ref, b1_ref,
                   w2t_ref, b2_ref, out_ref, pscr):
    P = out_ref.shape[1]

    C = x_ref.shape[2]

    def body(p, _):
        sl = sl_ref[0, 0, p]
        sc = sc_ref[0, 0, p]
        rows = x_ref[0, pl.ds(sl, _PL), :]
        rolled = pltpu.roll(rows, jax.lax.rem(C - sc, C), 1)
        pscr[:, pl.ds(p, 1), :] = rolled[:, :_PC].reshape(_PL, 1, _PC)
        return 0

    jax.lax.fori_loop(0, P, body, 0, unroll=2)

    slv = sl2_ref[0].astype(jnp.float32)
    iv = jax.lax.broadcasted_iota(jnp.int32, (P, _PL), 1).astype(jnp.float32)
    tv = (slv + iv) * (1.0 / _FS)
    hi = jax.lax.Precision.HIGHEST
    acc = jnp.dot(tv, w1t_ref[...], precision=hi) + b1_ref[...]
    for i in range(_PL):
        acc += jnp.dot(pscr[i], w1p_ref[i], precision=hi)
    h = acc * jax.nn.sigmoid(acc)
    out_ref[0] = jnp.dot(h, w2t_ref[...], precision=hi) + b2_ref[...]


def kernel(x, start_indices_L, start_indices_C, W1, b1, W2, b2):
    B, L, C = x.shape
    P = start_indices_L.shape[1]
    D = W2.shape[0]

    sl = start_indices_L.astype(jnp.int32).reshape(B, 1, P)
    sc = start_indices_C.astype(jnp.int32).reshape(B, 1, P)

    w1r = W1.reshape(D, _PL, 2 * _PC)
    w1p = w1r[:, :, :_PC].transpose(1, 2, 0)
    w1t = w1r[:, :, _PC:].sum(axis=2).T
    w2t = W2.T
    b1r = b1.reshape(1, D)
    b2r = b2.reshape(1, D)

    grid = (B,)
    out = pl.pallas_call(
        _mlp_tc_kernel,
        grid=grid,
        in_specs=[
            pl.BlockSpec((1, 1, P), lambda b: (b, 0, 0),
                         memory_space=pltpu.SMEM),
            pl.BlockSpec((1, 1, P), lambda b: (b, 0, 0),
                         memory_space=pltpu.SMEM),
            pl.BlockSpec((1, P, 1), lambda b: (b, 0, 0)),
            pl.BlockSpec((1, L, C), lambda b: (b, 0, 0)),
            pl.BlockSpec((_PL, _PC, D), lambda b: (0, 0, 0)),
            pl.BlockSpec((_PL, D), lambda b: (0, 0)),
            pl.BlockSpec((1, D), lambda b: (0, 0)),
            pl.BlockSpec((D, D), lambda b: (0, 0)),
            pl.BlockSpec((1, D), lambda b: (0, 0)),
        ],
        out_specs=pl.BlockSpec((1, P, D), lambda b: (b, 0, 0)),
        out_shape=jax.ShapeDtypeStruct((B, P, D), jnp.float32),
        scratch_shapes=[pltpu.VMEM((_PL, P, _PC), jnp.float32)],
    )(sl, sc, sl.reshape(B, P, 1), x, w1p, w1t, b1r, w2t, b2r)
    return out


# speedup vs baseline: 378.4476x; 3.2198x over previous
import functools

import jax
import jax.numpy as jnp
from jax import lax
from jax.experimental import pallas as pl
from jax.experimental.pallas import tpu as pltpu
from jax.experimental.pallas import tpu_sc as plsc

_PL, _PC = 8, 16
_FS = 100.0
_GR = 16
_NW = 32
_CHUNK = 128


def _sc_gather_body(L, C, xf_ref, sl_ref, sc_ref, out_ref,
                    slv, scv, eidx, patches, sem):
    num_cores = 2
    P = slv.shape[0]
    w = lax.axis_index("s") * num_cores + lax.axis_index("c")

    pltpu.sync_copy(sl_ref.at[pl.ds(w * P, P)], slv)
    pltpu.sync_copy(sc_ref.at[pl.ds(w * P, P)], scv)

    nchunks = P // 16
    base_w = w * (L * C)
    per_row = P // _CHUNK

    def build(c, _):
        sl16 = slv[pl.ds(c * 16, 16)]
        sc16 = scv[pl.ds(c * 16, 16)]
        base = base_w + sl16 * C + sc16
        col = (c % (_CHUNK // 16)) * 16
        rowoff = c // (_CHUNK // 16)
        for i in range(_PL):
            for j in range(_PC):
                k = i * _PC + j
                eidx[k * per_row + rowoff, pl.ds(col, 16)] = base + (i * C + j)
        return 0

    lax.fori_loop(0, nchunks, build, 0)

    ndma = (_PL * _PC * P) // _CHUNK
    group = 16
    handles = []
    for q in range(ndma):
        handles.append(pltpu.async_copy(
            xf_ref.at[eidx.at[q]],
            patches.at[pl.ds(q * _CHUNK, _CHUNK)], sem))
        if q >= group:
            handles[q - group].wait()
    for h in handles[ndma - group:]:
        h.wait()

    pltpu.sync_copy(patches, out_ref.at[w])


def _mlp_tc_kernel(sl2_ref, pt_ref, w1p_ref, w1t_ref, b1_ref,
                   w2t_ref, b2_ref, out_ref):
    P = out_ref.shape[1]
    slv = sl2_ref[0].astype(jnp.float32)
    iv = lax.broadcasted_iota(jnp.int32, (P, _PL), 1).astype(jnp.float32)
    tv = (slv + iv) * (1.0 / _FS)
    hi = lax.Precision.HIGHEST
    acc = jnp.dot(tv, w1t_ref[...], precision=hi) + b1_ref[...]
    acc += lax.dot_general(pt_ref[0], w1p_ref[...],
                           (((0,), (0,)), ((), ())), precision=hi)
    h = acc * jax.nn.sigmoid(acc)
    out_ref[0] = jnp.dot(h, w2t_ref[...], precision=hi) + b2_ref[...]


def kernel(x, start_indices_L, start_indices_C, W1, b1, W2, b2):
    B, L, C = x.shape
    P = start_indices_L.shape[1]
    D = W2.shape[0]
    BP = B * P

    sl = start_indices_L.astype(jnp.int32)
    sc = start_indices_C.astype(jnp.int32)
    xf = x.reshape(B * L * C)

    mesh = plsc.VectorSubcoreMesh(core_axis_name="c", subcore_axis_name="s")
    sc_gather = functools.partial(
        pl.kernel, mesh=mesh,
        out_type=jax.ShapeDtypeStruct((B, _PL * _PC * P), jnp.float32),
        scratch_types=[
            pltpu.VMEM((P,), jnp.int32),
            pltpu.VMEM((P,), jnp.int32),
            pltpu.VMEM((_PL * _PC * P // _CHUNK, _CHUNK), jnp.int32),
            pltpu.VMEM((_PL * _PC * P,), jnp.float32),
            pltpu.SemaphoreType.DMA,
        ],
    )(functools.partial(_sc_gather_body, L, C))
    pt = sc_gather(xf, sl.reshape(BP), sc.reshape(BP))
    pt = pt.reshape(B, _PL * _PC, P)

    w1r = W1.reshape(D, _PL, 2 * _PC)
    w1p = w1r[:, :, :_PC].reshape(D, _PL * _PC).T
    w1t = w1r[:, :, _PC:].sum(axis=2).T
    w2t = W2.T
    b1r = b1.reshape(1, D)
    b2r = b2.reshape(1, D)

    out = pl.pallas_call(
        _mlp_tc_kernel,
        grid=(B,),
        in_specs=[
            pl.BlockSpec((1, P, 1), lambda b: (b, 0, 0)),
            pl.BlockSpec((1, _PL * _PC, P), lambda b: (b, 0, 0)),
            pl.BlockSpec((_PL * _PC, D), lambda b: (0, 0)),
            pl.BlockSpec((_PL, D), lambda b: (0, 0)),
            pl.BlockSpec((1, D), lambda b: (0, 0)),
            pl.BlockSpec((D, D), lambda b: (0, 0)),
            pl.BlockSpec((1, D), lambda b: (0, 0)),
        ],
        out_specs=pl.BlockSpec((1, P, D), lambda b: (b, 0, 0)),
        out_shape=jax.ShapeDtypeStruct((B, P, D), jnp.float32),
    )(sl.reshape(B, P, 1), pt, w1p, w1t, b1r, w2t, b2r)
    return out
